# baseline (speedup 1.0000x reference)
import jax
import jax.numpy as jnp
from jax import lax
from jax.experimental import pallas as pl
from jax.experimental.pallas import tpu as pltpu
from jax.experimental.pallas import tpu_sc as plsc

BATCH = 16384
FIELDS = 26
VOCAB = 100000
DIM = 16
DENSE = 13
OUT_W = FIELDS * DIM + DENSE

NC = 2
NS = 16
NW = NC * NS
ROWS_PER_W = BATCH // NW
DEPTH = 13


def _body(xs_t_hbm, dense_hbm, tables_hbm, out_hbm, xs_all, slots, dense_v,
          gsems, wsem, dsem):
    wid = lax.axis_index("s") * NC + lax.axis_index("c")
    base = wid * ROWS_PER_W

    dense_in = pltpu.make_async_copy(
        dense_hbm.at[pl.ds(base, ROWS_PER_W), :], dense_v, dsem)
    dense_in.start()
    for f in range(FIELDS):
        pltpu.make_async_copy(
            xs_t_hbm.at[pl.ds(f * BATCH + base, ROWS_PER_W)],
            xs_all.at[pl.ds(f * ROWS_PER_W, ROWS_PER_W)],
            gsems[f % DEPTH],
        ).start()
    for f in range(FIELDS):
        pltpu.make_async_copy(
            xs_t_hbm.at[pl.ds(f * BATCH + base, ROWS_PER_W)],
            xs_all.at[pl.ds(f * ROWS_PER_W, ROWS_PER_W)],
            gsems[f % DEPTH],
        ).wait()
    dense_in.wait()
    dense_out = pltpu.make_async_copy(
        dense_v, out_hbm.at[pl.ds(base, ROWS_PER_W), pl.ds(FIELDS * DIM, DENSE)],
        dsem)
    dense_out.start()

    def gather(f):
        return pltpu.make_async_copy(
            tables_hbm.at[f].at[xs_all.at[pl.ds(f * ROWS_PER_W, ROWS_PER_W)]],
            slots[f % DEPTH],
            gsems[f % DEPTH],
        )

    def write(f):
        return pltpu.make_async_copy(
            slots[f % DEPTH],
            out_hbm.at[pl.ds(base, ROWS_PER_W), pl.ds(f * DIM, DIM)],
            wsem,
        )

    for f in range(DEPTH):
        gather(f).start()
    for f in range(FIELDS):
        if f >= 1 and f + DEPTH - 1 < FIELDS:
            write(f - 1).wait()
            gather(f + DEPTH - 1).start()
        gather(f).wait()
        write(f).start()
    for f in range(FIELDS - DEPTH, FIELDS):
        write(f).wait()
    dense_out.wait()


@jax.jit
def _run(xs_t_flat, x_dense, tables):
    mesh = plsc.VectorSubcoreMesh(
        core_axis_name="c", subcore_axis_name="s", num_cores=NC, num_subcores=NS
    )
    return pl.kernel(
        _body,
        out_type=jax.ShapeDtypeStruct((BATCH, OUT_W), jnp.float32),
        mesh=mesh,
        compiler_params=pltpu.CompilerParams(use_tc_tiling_on_sc=False),
        scratch_types=[
            pltpu.VMEM((FIELDS * ROWS_PER_W,), jnp.int32),
            [pltpu.VMEM((ROWS_PER_W, DIM), jnp.float32) for _ in range(DEPTH)],
            pltpu.VMEM((ROWS_PER_W, DENSE), jnp.float32),
            [pltpu.SemaphoreType.DMA for _ in range(DEPTH)],
            pltpu.SemaphoreType.DMA,
            pltpu.SemaphoreType.DMA,
        ],
    )(xs_t_flat, x_dense, tables)


def kernel(x_sparse, x_dense, tables):
    xs_t_flat = x_sparse.T.reshape(-1)
    return _run(xs_t_flat, x_dense, tables)
